# Initial kernel scaffold; baseline (speedup 1.0000x reference)
import jax
import jax.numpy as jnp
from jax.experimental import pallas as pl


def kernel(nxyz, bonded_nbr_list, embed, conv_W1, conv_b1, conv_W2, conv_b2, ro_W1, ro_b1, ro_W2, ro_b2, ro_W3, ro_b3, bond_W, bond_b):
    raise NotImplementedError("write your pallas kernel here")


# speedup vs baseline: 1.9191x; 1.9191x over previous
import functools

import jax
import jax.numpy as jnp
from jax import lax
from jax.experimental import pallas as pl
from jax.experimental.pallas import tpu as pltpu
from jax.experimental.pallas import tpu_sc as plsc

NC = 2
NS = 16
NW = NC * NS
C = 128

F32 = jnp.float32


def _sc_mesh():
    return plsc.VectorSubcoreMesh(core_axis_name="c", subcore_axis_name="s",
                                  num_cores=NC, num_subcores=NS)


def _embed_uv_call(nxyz, embed, A, B, blk):
    n = nxyz.shape[0]
    f = embed.shape[1]
    ne = embed.shape[0]

    def body(nxyz_ref, embed_ref, a_ref, b_ref, r_ref, u_ref, v_ref):
        z = nxyz_ref[:, 0].astype(jnp.int32)
        oh = (z[:, None] == lax.broadcasted_iota(jnp.int32, (blk, ne), 1))
        r = jnp.dot(oh.astype(F32), embed_ref[...], preferred_element_type=F32)
        r_ref[...] = r
        u_ref[...] = jnp.dot(r, a_ref[...], preferred_element_type=F32)
        v_ref[...] = jnp.dot(r, b_ref[...], preferred_element_type=F32)

    out = jax.ShapeDtypeStruct((n, f), F32)
    return pl.pallas_call(
        body,
        grid=(n // blk,),
        in_specs=[
            pl.BlockSpec((blk, 4), lambda i: (i, 0)),
            pl.BlockSpec((ne, f), lambda i: (0, 0)),
            pl.BlockSpec((f, f), lambda i: (0, 0)),
            pl.BlockSpec((f, f), lambda i: (0, 0)),
        ],
        out_specs=[pl.BlockSpec((blk, f), lambda i: (i, 0))] * 3,
        out_shape=[out, out, out],
    )(nxyz, embed, A, B)


def _uv_call(r, d0, d1, A, B, blk):
    n, f = r.shape

    def body(r_ref, d0_ref, d1_ref, a_ref, b_ref, ro_ref, u_ref, v_ref):
        rr = r_ref[...] + d0_ref[...] + d1_ref[...]
        ro_ref[...] = rr
        u_ref[...] = jnp.dot(rr, a_ref[...], preferred_element_type=F32)
        v_ref[...] = jnp.dot(rr, b_ref[...], preferred_element_type=F32)

    out = jax.ShapeDtypeStruct((n, f), F32)
    return pl.pallas_call(
        body,
        grid=(n // blk,),
        in_specs=[
            pl.BlockSpec((blk, f), lambda i: (i, 0)),
            pl.BlockSpec((blk, f), lambda i: (i, 0)),
            pl.BlockSpec((blk, f), lambda i: (i, 0)),
            pl.BlockSpec((f, f), lambda i: (0, 0)),
            pl.BlockSpec((f, f), lambda i: (0, 0)),
        ],
        out_specs=[pl.BlockSpec((blk, f), lambda i: (i, 0))] * 3,
        out_shape=[out, out, out],
    )(r, d0, d1, A, B)


def _edge_mlp_call(s, W2, b1, b2, blk):
    e, f = s.shape

    def body(s_ref, w_ref, b1_ref, b2_ref, m_ref):
        m1 = jnp.tanh(s_ref[...] + b1_ref[...])
        m2 = jnp.dot(m1, w_ref[...], preferred_element_type=F32) + b2_ref[...]
        m_ref[...] = jnp.tanh(m2)

    return pl.pallas_call(
        body,
        grid=(e // blk,),
        in_specs=[
            pl.BlockSpec((blk, f), lambda i: (i, 0)),
            pl.BlockSpec((f, f), lambda i: (0, 0)),
            pl.BlockSpec((1, f), lambda i: (0, 0)),
            pl.BlockSpec((1, f), lambda i: (0, 0)),
        ],
        out_specs=pl.BlockSpec((blk, f), lambda i: (i, 0)),
        out_shape=jax.ShapeDtypeStruct((e, f), F32),
    )(s, W2, b1, b2)


def _readout_call(r, d0, d1, roW1, rob1, roW2, rob2, roW3, rob3, bW, nxyz,
                  blk, n_real):
    n, f = r.shape
    nk = roW1.shape[0]
    h1 = roW1.shape[2]
    h2 = roW2.shape[2]

    def body(r_ref, d0_ref, d1_ref, w1_ref, c1_ref, w2_ref, c2_ref, w3_ref,
             c3_ref, bw_ref, nxyz_ref, t_ref, ea_ref):
        i = pl.program_id(0)
        rr = r_ref[...] + d0_ref[...] + d1_ref[...]
        rows = i * blk + lax.broadcasted_iota(jnp.int32, (blk, 1), 0)
        valid = rows < n_real
        sums = []
        qs = []
        for k in range(nk):
            h = jnp.tanh(jnp.dot(rr, w1_ref[k], preferred_element_type=F32)
                         + c1_ref[k][None, :])
            h = jnp.tanh(jnp.dot(h, w2_ref[k], preferred_element_type=F32)
                         + c2_ref[k][None, :])
            ea = jnp.dot(h, w3_ref[k], preferred_element_type=F32) + c3_ref[k][None, :]
            ea = jnp.where(valid, ea, jnp.float32(0.0))
            sums.append(jnp.sum(ea).reshape(1, 1))
            qs.append(jnp.dot(rr, bw_ref[k], preferred_element_type=F32))
        xyz = nxyz_ref[:, 1:4]
        pad = jnp.zeros((blk, f - 2 * nk - 3), F32)
        t_ref[...] = jnp.concatenate(qs + [xyz, pad], axis=1)
        val = jnp.concatenate(sums, axis=1)

        @pl.when(i == 0)
        def _():
            ea_ref[...] = val

        @pl.when(i != 0)
        def _():
            ea_ref[...] = ea_ref[...] + val

    return pl.pallas_call(
        body,
        grid=(n // blk,),
        in_specs=[
            pl.BlockSpec((blk, f), lambda i: (i, 0)),
            pl.BlockSpec((blk, f), lambda i: (i, 0)),
            pl.BlockSpec((blk, f), lambda i: (i, 0)),
            pl.BlockSpec((nk, f, h1), lambda i: (0, 0, 0)),
            pl.BlockSpec((nk, h1), lambda i: (0, 0)),
            pl.BlockSpec((nk, h1, h2), lambda i: (0, 0, 0)),
            pl.BlockSpec((nk, h2), lambda i: (0, 0)),
            pl.BlockSpec((nk, h2, 1), lambda i: (0, 0, 0)),
            pl.BlockSpec((nk, 1), lambda i: (0, 0)),
            pl.BlockSpec((nk, f, 2), lambda i: (0, 0, 0)),
            pl.BlockSpec((blk, 4), lambda i: (i, 0)),
        ],
        out_specs=[
            pl.BlockSpec((blk, f), lambda i: (i, 0)),
            pl.BlockSpec((1, nk), lambda i: (0, 0)),
        ],
        out_shape=[
            jax.ShapeDtypeStruct((n, f), F32),
            jax.ShapeDtypeStruct((1, nk), F32),
        ],
    )(r, d0, d1, roW1, rob1, roW2, rob2, roW3, rob3, bW, nxyz)


def _bond_reduce_call(ts, ea, bond_b, blk):
    e = ts.shape[0]
    nk = bond_b.shape[0]

    def body(ts_ref, ea_ref, bb_ref, out_ref):
        i = pl.program_id(0)
        t = ts_ref[...]
        w = t[:, 7:8]
        dvec = t[:, 4:7]
        dd = jnp.sqrt(jnp.sum(dvec * dvec, axis=1, keepdims=True) + 1e-12)
        parts = []
        for k in range(nk):
            bp = t[:, 2 * k:2 * k + 2] + bb_ref[k][None, :]
            kf = jax.nn.softplus(bp[:, 0:1])
            d0 = jax.nn.softplus(bp[:, 1:2])
            eb = kf * (dd - d0) ** 2 * w
            parts.append(jnp.sum(eb).reshape(1, 1))
        val = jnp.concatenate(parts, axis=1)

        @pl.when(i == 0)
        def _():
            out_ref[...] = ea_ref[...] + val

        @pl.when(i != 0)
        def _():
            out_ref[...] = out_ref[...] + val

    return pl.pallas_call(
        body,
        grid=(e // blk,),
        in_specs=[
            pl.BlockSpec((blk, 16), lambda i: (i, 0)),
            pl.BlockSpec((1, nk), lambda i: (0, 0)),
            pl.BlockSpec((nk, nk), lambda i: (0, 0)),
        ],
        out_specs=pl.BlockSpec((1, nk), lambda i: (0, 0)),
        out_shape=jax.ShapeDtypeStruct((1, nk), F32),
    )(ts, ea, bond_b)


def _gather_add_call(src2, dst2, u, v):
    n, f = u.shape
    nch_all, c = src2.shape
    nch = nch_all // NW
    ew = nch * c

    @functools.partial(
        pl.kernel,
        out_type=jax.ShapeDtypeStruct((nch_all * c, f), F32),
        mesh=_sc_mesh(),
        scratch_types=[
            pltpu.VMEM((nch, c), jnp.int32),
            pltpu.VMEM((nch, c), jnp.int32),
            pltpu.VMEM((c, f), F32),
            pltpu.VMEM((c, f), F32),
            pltpu.SemaphoreType.DMA,
        ],
    )
    def k(src_hbm, dst_hbm, u_hbm, v_hbm, out_hbm, si_v, di_v, ru_v, rv_v, sem):
        wid = lax.axis_index("s") * NC + lax.axis_index("c")
        pltpu.sync_copy(src_hbm.at[pl.ds(wid * nch, nch)], si_v)
        pltpu.sync_copy(dst_hbm.at[pl.ds(wid * nch, nch)], di_v)

        def chunk(ci, _):
            off = wid * ew + ci * c
            cp1 = pltpu.async_copy(u_hbm.at[si_v.at[ci]], ru_v, sem)
            cp2 = pltpu.async_copy(v_hbm.at[di_v.at[ci]], rv_v, sem)
            cp1.wait()
            cp2.wait()

            def row(j, _):
                for l in range(f // 16):
                    sl = pl.ds(l * 16, 16)
                    ru_v[j, sl] = ru_v[j, sl] + rv_v[j, sl]
                return 0

            lax.fori_loop(0, c, row, 0)
            pltpu.sync_copy(ru_v, out_hbm.at[pl.ds(off, c)])
            return 0

        lax.fori_loop(0, nch, chunk, 0)

    return k(src2, dst2, u, v)


def _scatter_add_call(m, dst2, n):
    e, f = m.shape
    nch_all, c = dst2.shape
    nch = nch_all // NW
    ew = nch * c
    rpt = n // NS
    zr = 128

    @functools.partial(
        pl.kernel,
        out_type=jax.ShapeDtypeStruct((NC, n, f), F32),
        mesh=_sc_mesh(),
        scratch_types=[
            pltpu.VMEM((c, f), F32),
            pltpu.VMEM((nch, c), jnp.int32),
            pltpu.VMEM((zr, f), F32),
            pltpu.VMEM_SHARED((n, f), F32),
        ],
    )
    def k(m_hbm, dst_hbm, out_hbm, rows_v, di_v, z_v, acc_sh):
        cid = lax.axis_index("c")
        sid = lax.axis_index("s")
        wid = sid * NC + cid

        def zrow(j, _):
            for l in range(f // 16):
                z_v[j, pl.ds(l * 16, 16)] = jnp.zeros((16,), F32)
            return 0

        lax.fori_loop(0, zr, zrow, 0)
        for b in range(rpt // zr):
            pltpu.sync_copy(z_v, acc_sh.at[pl.ds(sid * rpt + b * zr, zr)])
        pltpu.sync_copy(dst_hbm.at[pl.ds(wid * nch, nch)], di_v)
        plsc.subcore_barrier()

        def chunk(ci, _):
            off = wid * ew + ci * c
            pltpu.sync_copy(m_hbm.at[pl.ds(off, c)], rows_v)
            pltpu.sync_copy(rows_v, acc_sh.at[di_v.at[ci]], add=True)
            return 0

        lax.fori_loop(0, nch, chunk, 0)
        plsc.subcore_barrier()
        pltpu.sync_copy(acc_sh.at[pl.ds(sid * rpt, rpt)],
                        out_hbm.at[cid, pl.ds(sid * rpt, rpt)])

    return k(m, dst2)


def _bond_gather_call(src2, dst2, t, ew_valid):
    n, f = t.shape
    nch_all, c = src2.shape
    nch = nch_all // NW
    ew = nch * c

    @functools.partial(
        pl.kernel,
        out_type=jax.ShapeDtypeStruct((nch_all * c, 16), F32),
        mesh=_sc_mesh(),
        scratch_types=[
            pltpu.VMEM((nch, c), jnp.int32),
            pltpu.VMEM((nch, c), jnp.int32),
            pltpu.VMEM((c, f), F32),
            pltpu.VMEM((c, f), F32),
            pltpu.VMEM((c, 16), F32),
            pltpu.SemaphoreType.DMA,
        ],
    )
    def k(src_hbm, dst_hbm, t_hbm, out_hbm, si_v, di_v, ta_v, tb_v, ob_v, sem):
        wid = lax.axis_index("s") * NC + lax.axis_index("c")
        pltpu.sync_copy(src_hbm.at[pl.ds(wid * nch, nch)], si_v)
        pltpu.sync_copy(dst_hbm.at[pl.ds(wid * nch, nch)], di_v)
        lane = lax.iota(jnp.int32, 16)
        sgn = jnp.where(lane < 4, jnp.float32(1.0), jnp.float32(-1.0))
        mark = jnp.where(lane == 7, jnp.float32(1.0), jnp.float32(0.0))

        def chunk(ci, _):
            off = wid * ew + ci * c
            cp1 = pltpu.async_copy(t_hbm.at[si_v.at[ci]], ta_v, sem)
            cp2 = pltpu.async_copy(t_hbm.at[di_v.at[ci]], tb_v, sem)
            cp1.wait()
            cp2.wait()

            def row(j, _):
                sl = pl.ds(0, 16)
                w = jnp.where(ci * c + j < ew_valid, jnp.float32(1.0),
                              jnp.float32(0.0))
                ob_v[j, sl] = (ta_v[j, sl] + tb_v[j, sl] * sgn + mark) * w
                return 0

            lax.fori_loop(0, c, row, 0)
            pltpu.sync_copy(ob_v, out_hbm.at[pl.ds(off, c)])
            return 0

        lax.fori_loop(0, nch, chunk, 0)

    return k(src2, dst2, t)


def kernel(nxyz, bonded_nbr_list, embed, conv_W1, conv_b1, conv_W2, conv_b2,
           ro_W1, ro_b1, ro_W2, ro_b2, ro_W3, ro_b3, bond_W, bond_b):
    n = nxyz.shape[0]
    e = bonded_nbr_list.shape[0]
    f = embed.shape[1]

    ew_valid = e // NW
    ew = ((ew_valid + C - 1) // C + 7) // 8 * 8 * C
    n_pad = -(-n // (NS * 128)) * NS * 128

    src = bonded_nbr_list[:, 0].astype(jnp.int32).reshape(NW, ew_valid)
    dst = bonded_nbr_list[:, 1].astype(jnp.int32).reshape(NW, ew_valid)
    fill = jnp.full((NW, ew - ew_valid), n, jnp.int32)
    src2 = jnp.concatenate([src, fill], axis=1).reshape(NW * ew // C, C)
    dst2 = jnp.concatenate([dst, fill], axis=1).reshape(NW * ew // C, C)
    npad_rows = jnp.concatenate(
        [jnp.ones((n_pad - n, 1), F32), jnp.zeros((n_pad - n, 3), F32)], axis=1)
    nxyz_p = jnp.concatenate([nxyz.astype(F32), npad_rows], axis=0)

    nblk = n_pad // 10
    eblk = 2048

    d0 = d1 = None
    r = u = v = None
    for i in range(conv_W1.shape[0]):
        A = conv_W1[i, :f]
        B = conv_W1[i, f:]
        if i == 0:
            r, u, v = _embed_uv_call(nxyz_p, embed, A, B, nblk)
        else:
            r, u, v = _uv_call(r, d0, d1, A, B, nblk)
        s = _gather_add_call(src2, dst2, u, v)
        m = _edge_mlp_call(s, conv_W2[i], conv_b1[i].reshape(1, f),
                           conv_b2[i].reshape(1, f), eblk)
        drp = _scatter_add_call(m, dst2, n_pad)
        d0, d1 = drp[0], drp[1]

    t, ea = _readout_call(r, d0, d1, ro_W1, ro_b1, ro_W2, ro_b2, ro_W3, ro_b3,
                          bond_W, nxyz_p, nblk, n)
    ts = _bond_gather_call(src2, dst2, t, ew_valid)
    out = _bond_reduce_call(ts, ea, bond_b, 4096)
    return out.reshape(bond_b.shape[0])
